# Initial kernel scaffold; baseline (speedup 1.0000x reference)
import functools

import jax
import jax.numpy as jnp
from jax import lax
from jax.experimental import pallas as pl
from jax.experimental.pallas import tpu as pltpu
from jax.experimental.pallas import tpu_sc as plsc

NC = 2
NS = 16
NW = NC * NS

CHUNK = 1024
SUB = 128


def kernel(sentence, word_embeddings):
    B, S = sentence.shape
    D = word_embeddings.shape[1]
    n = B * S
    idx = sentence.reshape(n).astype(jnp.int32)
    per_w = n // NW
    n_chunks = per_w // CHUNK

    mesh = plsc.VectorSubcoreMesh(
        core_axis_name="c", subcore_axis_name="s",
        num_cores=NC, num_subcores=NS,
    )

    @functools.partial(
        pl.kernel,
        out_type=jax.ShapeDtypeStruct((n, D), jnp.float32),
        mesh=mesh,
        scratch_types=[
            pltpu.VMEM((CHUNK,), jnp.int32),
            pltpu.VMEM((CHUNK, D), jnp.float32),
            pltpu.SemaphoreType.DMA,
        ],
    )
    def run(idx_hbm, tab_hbm, out_hbm, idx_v, rows_v, sem):
        wid = lax.axis_index("s") * NC + lax.axis_index("c")
        base_w = wid * per_w

        def body(c, carry):
            base = base_w + c * CHUNK
            pltpu.sync_copy(idx_hbm.at[pl.ds(base, CHUNK)], idx_v)
            handles = []
            for j in range(CHUNK // SUB):
                handles.append(pltpu.async_copy(
                    tab_hbm.at[idx_v.at[pl.ds(j * SUB, SUB)]],
                    rows_v.at[pl.ds(j * SUB, SUB)],
                    sem,
                ))
            for h in handles:
                h.wait()
            pltpu.sync_copy(rows_v, out_hbm.at[pl.ds(base, CHUNK)])
            return carry

        lax.fori_loop(0, n_chunks, body, 0)

    out = run(idx, word_embeddings)
    return out.reshape(B, S, D)


# speedup vs baseline: 1.4576x; 1.4576x over previous
import functools

import jax
import jax.numpy as jnp
from jax import lax
from jax.experimental import pallas as pl
from jax.experimental.pallas import tpu as pltpu
from jax.experimental.pallas import tpu_sc as plsc

NC = 2
NS = 16
NW = NC * NS

CHUNK = 1024
SUB = 128


def kernel(sentence, word_embeddings):
    B, S = sentence.shape
    D = word_embeddings.shape[1]
    n = B * S
    idx = sentence.reshape(n).astype(jnp.int32)
    per_w = n // NW
    n_chunks = per_w // CHUNK

    mesh = plsc.VectorSubcoreMesh(
        core_axis_name="c", subcore_axis_name="s",
        num_cores=NC, num_subcores=NS,
    )

    @functools.partial(
        pl.kernel,
        out_type=jax.ShapeDtypeStruct((n, D), jnp.float32),
        mesh=mesh,
        scratch_types=[
            pltpu.VMEM((CHUNK,), jnp.int32),
            pltpu.VMEM((CHUNK, D), jnp.float32),
            pltpu.SemaphoreType.DMA,
        ],
        compiler_params=pltpu.CompilerParams(use_tc_tiling_on_sc=False),
    )
    def run(idx_hbm, tab_hbm, out_hbm, idx_v, rows_v, sem):
        wid = lax.axis_index("s") * NC + lax.axis_index("c")
        base_w = wid * per_w

        def body(c, carry):
            base = base_w + c * CHUNK
            pltpu.sync_copy(idx_hbm.at[pl.ds(base, CHUNK)], idx_v)
            handles = []
            for j in range(CHUNK // SUB):
                handles.append(pltpu.async_copy(
                    tab_hbm.at[idx_v.at[pl.ds(j * SUB, SUB)]],
                    rows_v.at[pl.ds(j * SUB, SUB)],
                    sem,
                ))
            for h in handles:
                h.wait()
            pltpu.sync_copy(rows_v, out_hbm.at[pl.ds(base, CHUNK)])
            return carry

        lax.fori_loop(0, n_chunks, body, 0)

    out = run(idx, word_embeddings)
    return out.reshape(B, S, D)


# speedup vs baseline: 1.4584x; 1.0005x over previous
import functools

import jax
import jax.numpy as jnp
from jax import lax
from jax.experimental import pallas as pl
from jax.experimental.pallas import tpu as pltpu
from jax.experimental.pallas import tpu_sc as plsc

NC = 2
NS = 16
NW = NC * NS

CHUNK = 1024
SUB = 128


def kernel(sentence, word_embeddings):
    B, S = sentence.shape
    D = word_embeddings.shape[1]
    n = B * S
    idx = sentence.reshape(n).astype(jnp.int32)
    per_w = n // NW
    n_chunks = per_w // CHUNK

    mesh = plsc.VectorSubcoreMesh(
        core_axis_name="c", subcore_axis_name="s",
        num_cores=NC, num_subcores=NS,
    )

    @functools.partial(
        pl.kernel,
        out_type=jax.ShapeDtypeStruct((n, D), jnp.float32),
        mesh=mesh,
        scratch_types=[
            pltpu.VMEM((CHUNK,), jnp.int32),
            pltpu.VMEM((CHUNK, D), jnp.float32),
            pltpu.SemaphoreType.DMA,
        ],
        compiler_params=pltpu.CompilerParams(use_tc_tiling_on_sc=False),
    )
    def run(idx_hbm, tab_hbm, out_hbm, idx_v, rows_v, sem):
        wid = lax.axis_index("s") * NC + lax.axis_index("c")
        base_w = wid * per_w

        def body(c, carry):
            base = base_w + c * CHUNK
            pltpu.sync_copy(idx_hbm.at[pl.ds(base, CHUNK)], idx_v)
            pltpu.async_copy(tab_hbm.at[idx_v], rows_v, sem).wait()
            pltpu.sync_copy(rows_v, out_hbm.at[pl.ds(base, CHUNK)])
            return carry

        lax.fori_loop(0, n_chunks, body, 0)

    out = run(idx, word_embeddings)
    return out.reshape(B, S, D)


# speedup vs baseline: 1.4908x; 1.0222x over previous
import functools

import jax
import jax.numpy as jnp
from jax import lax
from jax.experimental import pallas as pl
from jax.experimental.pallas import tpu as pltpu
from jax.experimental.pallas import tpu_sc as plsc

NC = 2
NS = 16
NW = NC * NS

CHUNK = 800
RING = 4


def kernel(sentence, word_embeddings):
    B, S = sentence.shape
    D = word_embeddings.shape[1]
    n = B * S
    idx = sentence.reshape(n).astype(jnp.int32)
    per_w = n // NW
    n_chunks = per_w // CHUNK
    n_iters = n_chunks // RING

    mesh = plsc.VectorSubcoreMesh(
        core_axis_name="c", subcore_axis_name="s",
        num_cores=NC, num_subcores=NS,
    )

    @functools.partial(
        pl.kernel,
        out_type=jax.ShapeDtypeStruct((n, D), jnp.float32),
        mesh=mesh,
        scratch_types=[
            pltpu.VMEM((RING * CHUNK,), jnp.int32),
            pltpu.VMEM((RING * CHUNK, D), jnp.float32),
            pltpu.SemaphoreType.DMA((RING,)),
            pltpu.SemaphoreType.DMA((RING,)),
            pltpu.SemaphoreType.DMA((RING,)),
        ],
        compiler_params=pltpu.CompilerParams(use_tc_tiling_on_sc=False),
    )
    def run(idx_hbm, tab_hbm, out_hbm, idx_v, rows_v, isem, gsem, ssem):
        wid = lax.axis_index("s") * NC + lax.axis_index("c")
        base_w = wid * per_w

        def body(g, carry):
            base = base_w + g * (RING * CHUNK)
            iload = []
            for b in range(RING):
                iload.append(pltpu.async_copy(
                    idx_hbm.at[pl.ds(base + b * CHUNK, CHUNK)],
                    idx_v.at[pl.ds(b * CHUNK, CHUNK)], isem.at[b],
                ))
            gath = []
            for b in range(RING):
                iload[b].wait()
                gath.append(pltpu.async_copy(
                    tab_hbm.at[idx_v.at[pl.ds(b * CHUNK, CHUNK)]],
                    rows_v.at[pl.ds(b * CHUNK, CHUNK)], gsem.at[b],
                ))
            store = []
            for b in range(RING):
                gath[b].wait()
                store.append(pltpu.async_copy(
                    rows_v.at[pl.ds(b * CHUNK, CHUNK)],
                    out_hbm.at[pl.ds(base + b * CHUNK, CHUNK)],
                    ssem.at[b],
                ))
            for b in range(RING):
                store[b].wait()
            return carry

        lax.fori_loop(0, n_iters, body, 0)

    out = run(idx, word_embeddings)
    return out.reshape(B, S, D)
